# baseline (speedup 1.0000x reference)
import functools
import math

import numpy as np
import jax
import jax.numpy as jnp
from jax import lax
from jax.experimental import pallas as pl
from jax.experimental.pallas import tpu as pltpu
from jax.experimental.pallas import tpu_sc as plsc

_B, _S, _E = 4096, 200, 64
_NW = 32
_ROWS_PER_W = _B // _NW
_VP = 500000

_GROUPS = tuple(list(range(0, 192, 16)) + [184])


def _pos_encoding_np(seq_len=_S, emb_size=_E):
    position_idx = np.arange(0, seq_len, dtype=np.float32)[:, None]
    fill = position_idx * np.exp(
        -np.arange(0, emb_size, 2, dtype=np.float32) * math.log(10000.0) / emb_size)
    pos = np.zeros((seq_len, emb_size), dtype=np.float32)
    pos[:, 0::2] = np.sin(fill)
    pos[:, 1::2] = np.cos(fill)
    return pos


_POS_T_FLAT = np.ascontiguousarray(_pos_encoding_np().T).reshape(-1)


def _make_kernel():
    mesh = plsc.VectorSubcoreMesh(core_axis_name="c", subcore_axis_name="s")

    scratch = (
        [pltpu.VMEM((_S,), jnp.int32)] * 3
        + [pltpu.VMEM((_S,), jnp.int32)] * 3
        + [pltpu.VMEM((_S,), jnp.int32)] * 3
        + [pltpu.VMEM((_S, 2 * _E), jnp.float32)] * 3
        + [pltpu.VMEM((_S, _E), jnp.float32)] * 2
        + [pltpu.VMEM((_E * _S,), jnp.float32)]
        + [pltpu.SemaphoreType.DMA] * 3
        + [pltpu.SemaphoreType.DMA] * 3
        + [pltpu.SemaphoreType.DMA] * 2
    )

    @functools.partial(
        pl.kernel,
        out_type=jax.ShapeDtypeStruct((_B, _S, 2 * _E), jnp.float32),
        mesh=mesh,
        scratch_types=scratch,
        compiler_params=pltpu.CompilerParams(use_tc_tiling_on_sc=False,
                                             needs_layout_passes=False),
    )
    def emb_kernel(x_hbm, table_hbm, post_hbm, out_hbm,
                   i0, i1, i2, p0, p1, p2, q0, q1, q2,
                   r0, r1, r2, o0, o1, post_v,
                   si0, si1, si2, sg0, sg1, sg2, sw0, sw1):
        idxr = (i0, i1, i2)
        pairr = (p0, p1, p2)
        parr = (q0, q1, q2)
        rows = (r0, r1, r2)
        res = (o0, o1)
        isem = (si0, si1, si2)
        gsem = (sg0, sg1, sg2)
        wsem = (sw0, sw1)

        wid = lax.axis_index("s") * 2 + lax.axis_index("c")
        base = wid * _ROWS_PER_W

        pltpu.sync_copy(post_hbm, post_v)

        def start_idx(v, b):
            pltpu.async_copy(x_hbm.at[pl.ds((base + v) * _S, _S)],
                             idxr[b], isem[b])

        def wait_idx(b):
            pltpu.make_async_copy(x_hbm.at[pl.ds(0, _S)], idxr[b],
                                  isem[b]).wait()

        def transform(b):
            def tbody(g, c):
                r = jnp.minimum(g * 16, _S - 16)
                sl = pl.ds(r, 16)
                iv = idxr[b][sl]
                pairr[b][sl] = jax.lax.shift_right_logical(iv, 1)
                parr[b][sl] = jax.lax.bitwise_and(iv, 1)
                return c

            lax.fori_loop(0, len(_GROUPS), tbody, 0)

        def start_gather(b):
            pltpu.async_copy(table_hbm.at[pairr[b]], rows[b], gsem[b])

        def wait_gather(b):
            pltpu.make_async_copy(table_hbm.at[pairr[0]], rows[b],
                                  gsem[b]).wait()

        def wait_write(b):
            pltpu.make_async_copy(res[b], out_hbm.at[base, :, pl.ds(0, _E)],
                                  wsem[b]).wait()

        def select_add(b3, b2):
            rbuf = rows[b3]
            obuf = res[b2]

            def sbody(g, c):
                r = jnp.minimum(g * 16, _S - 16)
                rowvec = jnp.arange(16, dtype=jnp.int32) + r
                colbase = jax.lax.shift_left(parr[b3][pl.ds(r, 16)], 6)

                @plsc.parallel_loop(0, _E, unroll=4)
                def _(col):
                    colv = colbase + col
                    colfull = jnp.full((16,), 1, jnp.int32) * col
                    val = plsc.load_gather(rbuf, [rowvec, colv])
                    pv = post_v[pl.ds(col * _S + r, 16)]
                    plsc.store_scatter(obuf, [rowvec, colfull], val + pv)
                return c

            lax.fori_loop(0, len(_GROUPS), sbody, 0)

        def visit(v, b3, b2, wwait=True, do_a=True, do_b=True):
            if do_a:
                start_idx(v + 2, (b3 + 2) % 3)
            if do_b:
                bn = (b3 + 1) % 3
                wait_idx(bn)
                transform(bn)
                start_gather(bn)
            wait_gather(b3)
            if wwait:
                wait_write(b2)
            select_add(b3, b2)
            pltpu.async_copy(res[b2], out_hbm.at[base + v, :, pl.ds(0, _E)],
                             wsem[b2])

        start_idx(0, 0)
        start_idx(1, 1)
        wait_idx(0)
        transform(0)
        start_gather(0)

        for j in range(6):
            visit(j, j % 3, j % 2, wwait=(j >= 2))

        def body(k, carry):
            v0 = 6 * k + 6
            for j in range(6):
                visit(v0 + j, j % 3, j % 2)
            return carry

        lax.fori_loop(0, 20, body, 0)

        visit(126, 0, 0, do_a=False, do_b=True)
        visit(127, 1, 1, do_a=False, do_b=False)

        wait_write(0)
        wait_write(1)

    return emb_kernel


_EMB_KERNEL = _make_kernel()


@jax.jit
def kernel(x, emb_table):
    post = jnp.asarray(_POS_T_FLAT)
    pair_view = jnp.reshape(emb_table, (_VP, 2 * _E))
    out128 = _EMB_KERNEL(x.astype(jnp.int32).reshape(-1), pair_view, post)
    return out128[:, :, :_E]


# speedup vs baseline: 1.8153x; 1.8153x over previous
import functools
import math

import numpy as np
import jax
import jax.numpy as jnp
from jax import lax
from jax.experimental import pallas as pl
from jax.experimental.pallas import tpu as pltpu
from jax.experimental.pallas import tpu_sc as plsc

_B, _S, _E = 4096, 200, 64
_NW = 32
_ROWS_PER_W = _B // _NW


def _pos_encoding_np(seq_len=_S, emb_size=_E):
    position_idx = np.arange(0, seq_len, dtype=np.float32)[:, None]
    fill = position_idx * np.exp(
        -np.arange(0, emb_size, 2, dtype=np.float32) * math.log(10000.0) / emb_size)
    pos = np.zeros((seq_len, emb_size), dtype=np.float32)
    pos[:, 0::2] = np.sin(fill)
    pos[:, 1::2] = np.cos(fill)
    return pos


_POS_FLAT = _pos_encoding_np().reshape(-1)


def _make_kernel():
    mesh = plsc.VectorSubcoreMesh(core_axis_name="c", subcore_axis_name="s")

    scratch = (
        [pltpu.VMEM((_S,), jnp.int32)] * 3
        + [pltpu.VMEM((_S, 2 * _E), jnp.float32)] * 2
        + [pltpu.VMEM((_S, _E), jnp.float32)] * 2
        + [pltpu.VMEM((_S * _E,), jnp.float32)]
        + [pltpu.SemaphoreType.DMA] * 3
        + [pltpu.SemaphoreType.DMA] * 2
        + [pltpu.SemaphoreType.DMA] * 2
    )

    @functools.partial(
        pl.kernel,
        out_type=jax.ShapeDtypeStruct((_B, _S, _E), jnp.float32),
        mesh=mesh,
        scratch_types=scratch,
    )
    def emb_kernel(x_hbm, table_hbm, pos_hbm, out_hbm,
                   i0, i1, i2, r0, r1, o0, o1, pos_v,
                   si0, si1, si2, g0, g1, w0, w1):
        idxr = (i0, i1, i2)
        isem = (si0, si1, si2)
        rows = (r0, r1)
        res = (o0, o1)
        gsem = (g0, g1)
        wsem = (w0, w1)

        wid = lax.axis_index("s") * 2 + lax.axis_index("c")
        base = wid * _ROWS_PER_W

        pltpu.sync_copy(pos_hbm, pos_v)

        def start_idx(v, bi):
            pltpu.async_copy(x_hbm.at[pl.ds((base + v) * _S, _S)],
                             idxr[bi], isem[bi])

        def wait_idx(bi):
            pltpu.make_async_copy(x_hbm.at[pl.ds(0, _S)], idxr[bi],
                                  isem[bi]).wait()

        def start_gather(v, b, bi):
            pltpu.async_copy(table_hbm.at[idxr[bi]], rows[b], gsem[b])

        def wait_gather(b):
            pltpu.make_async_copy(table_hbm.at[idxr[0]], rows[b],
                                  gsem[b]).wait()

        def wait_write(b):
            pltpu.make_async_copy(res[b], out_hbm.at[base], wsem[b]).wait()

        def add_pos(b):
            rbuf = rows[b]
            obuf = res[b]

            @plsc.parallel_loop(0, _S, unroll=4)
            def _(r):
                for c in range(_E // 16):
                    sl = pl.ds(c * 16, 16)
                    obuf[r, sl] = rbuf[r, sl] + pos_v[pl.ds(r * _E + c * 16, 16)]

        def visit(v, b, bi, wwait=True, do_a=True, do_g=True):
            if do_a:
                start_idx(v + 2, (bi + 2) % 3)
            if do_g:
                wait_idx((bi + 1) % 3)
                start_gather(v + 1, (b + 1) % 2, (bi + 1) % 3)
            wait_gather(b)
            if wwait:
                wait_write(b)
            add_pos(b)
            pltpu.async_copy(res[b], out_hbm.at[base + v], wsem[b])

        start_idx(0, 0)
        start_idx(1, 1)
        wait_idx(0)
        start_gather(0, 0, 0)

        for j in range(6):
            visit(j, j % 2, j % 3, wwait=(j >= 2))

        def body(k, carry):
            v0 = 6 * k + 6
            for j in range(6):
                visit(v0 + j, j % 2, j % 3)
            return carry

        lax.fori_loop(0, 19, body, 0)

        for j in range(8):
            v = 120 + j
            visit(v, j % 2, j % 3, do_a=(v + 2 < _ROWS_PER_W),
                  do_g=(v + 1 < _ROWS_PER_W))

        wait_write(0)
        wait_write(1)

    return emb_kernel


_EMB_KERNEL = _make_kernel()


@jax.jit
def kernel(x, emb_table):
    pos = jnp.asarray(_POS_FLAT)
    table128 = jnp.pad(emb_table, ((0, 0), (0, _E)))
    return _EMB_KERNEL(x.astype(jnp.int32).reshape(-1), table128, pos)


# speedup vs baseline: 2.1435x; 1.1808x over previous
import functools
import math

import numpy as np
import jax
import jax.numpy as jnp
from jax import lax
from jax.experimental import pallas as pl
from jax.experimental.pallas import tpu as pltpu
from jax.experimental.pallas import tpu_sc as plsc

_B, _S, _E = 4096, 200, 64
_NW = 32
_ROWS_PER_W = _B // _NW
_NBUF = 4
_AHEAD = 2


def _pos_encoding_np(seq_len=_S, emb_size=_E):
    position_idx = np.arange(0, seq_len, dtype=np.float32)[:, None]
    fill = position_idx * np.exp(
        -np.arange(0, emb_size, 2, dtype=np.float32) * math.log(10000.0) / emb_size)
    pos = np.zeros((seq_len, emb_size), dtype=np.float32)
    pos[:, 0::2] = np.sin(fill)
    pos[:, 1::2] = np.cos(fill)
    return pos


_POS = _pos_encoding_np()


def _make_kernel():
    mesh = plsc.VectorSubcoreMesh(core_axis_name="c", subcore_axis_name="s")

    row_buf = pltpu.VMEM((_S, _E), jnp.float32)
    scratch = (
        [pltpu.VMEM((_ROWS_PER_W, _S), jnp.int32)]
        + [pltpu.VMEM((_S, _E), jnp.float32)]
        + [row_buf] * _NBUF
        + [pltpu.SemaphoreType.DMA] * _NBUF
        + [pltpu.SemaphoreType.DMA] * _NBUF
    )

    @functools.partial(
        pl.kernel,
        out_type=jax.ShapeDtypeStruct((_B, _S, 2 * _E), jnp.float32),
        mesh=mesh,
        scratch_types=scratch,
        compiler_params=pltpu.CompilerParams(use_tc_tiling_on_sc=False),
    )
    def emb_kernel(x_hbm, table_hbm, pos_hbm, out_hbm, idx_all, pos_v,
                   r0, r1, r2, r3, g0, g1, g2, g3, w0, w1, w2, w3):
        rows = (r0, r1, r2, r3)
        gsem = (g0, g1, g2, g3)
        wsem = (w0, w1, w2, w3)

        wid = lax.axis_index("s") * 2 + lax.axis_index("c")
        base = wid * _ROWS_PER_W

        pltpu.sync_copy(pos_hbm, pos_v)
        pltpu.sync_copy(x_hbm.at[pl.ds(base, _ROWS_PER_W)], idx_all)

        def start_gather(v, b):
            pltpu.async_copy(table_hbm.at[idx_all.at[v]], rows[b], gsem[b])

        def wait_gather(b):
            pltpu.make_async_copy(table_hbm.at[idx_all.at[0]], rows[b],
                                  gsem[b]).wait()

        def wait_write(b):
            pltpu.make_async_copy(rows[b], out_hbm.at[base, :, pl.ds(0, _E)],
                                  wsem[b]).wait()

        def add_pos(b):
            buf = rows[b]

            @plsc.parallel_loop(0, _S, unroll=4)
            def _(r):
                for c in range(_E // 16):
                    sl = pl.ds(c * 16, 16)
                    buf[r, sl] = buf[r, sl] + pos_v[r, sl]

        def visit(v, b, do_wwait, do_gather):
            wait_gather(b)
            add_pos(b)
            pltpu.async_copy(rows[b], out_hbm.at[base + v, :, pl.ds(0, _E)],
                             wsem[b])
            if do_gather:
                bn = (b + _AHEAD) % _NBUF
                if do_wwait:
                    wait_write(bn)
                start_gather(v + _AHEAD, bn)

        start_gather(0, 0)
        start_gather(1, 1)

        for b in range(_NBUF):
            visit(b, b, do_wwait=(b >= _AHEAD), do_gather=True)

        def body(k, carry):
            for b in range(_NBUF):
                visit(_NBUF * k + b, b, do_wwait=True, do_gather=True)
            return carry

        lax.fori_loop(1, _ROWS_PER_W // _NBUF - 1, body, 0)

        last = _ROWS_PER_W - _NBUF
        for b in range(_NBUF):
            visit(last + b, b, do_wwait=(b < _AHEAD), do_gather=(b < _AHEAD))

        for b in range(_NBUF):
            wait_write(b)

    return emb_kernel


_EMB_KERNEL = _make_kernel()


@jax.jit
def kernel(x, emb_table):
    pos = jnp.asarray(_POS)
    out128 = _EMB_KERNEL(x.astype(jnp.int32), emb_table, pos)
    return out128[:, :, :_E]
